# baseline (speedup 1.0000x reference)
import functools

import jax
import jax.numpy as jnp
from jax import lax
from jax.experimental import pallas as pl
from jax.experimental.pallas import tpu as pltpu
from jax.experimental.pallas import tpu_sc as plsc

_INFO = plsc.get_sparse_core_info()
_NC = _INFO.num_cores
_NS = _INFO.num_subcores
_NW = _NC * _NS

_B_SC = 10240
_CHUNK = 40
_NBUF = 3

_B_BLK = 512
_K_BLK = 512


def _make_sc_gather(batch_total: int, d_model: int):
    b_per_w = _B_SC // _NW
    nch = b_per_w // _CHUNK

    mesh = plsc.VectorSubcoreMesh(core_axis_name="c", subcore_axis_name="s")

    @functools.partial(
        pl.kernel,
        mesh=mesh,
        compiler_params=pltpu.CompilerParams(use_tc_tiling_on_sc=True),
        out_type=jax.ShapeDtypeStruct((batch_total, d_model), jnp.float32),
        scratch_types=[
            pltpu.VMEM((nch, _CHUNK), jnp.int32),
            [pltpu.VMEM((_CHUNK, d_model), jnp.float32)] * _NBUF,
            [pltpu.SemaphoreType.DMA] * _NBUF,
            [pltpu.SemaphoreType.DMA] * _NBUF,
        ],
    )
    def gather_kernel(idx_hbm, table_hbm, out_hbm, idx_v, bufs, gsems,
                      osems):
        wid = lax.axis_index("s") * _NC + lax.axis_index("c")
        pltpu.sync_copy(idx_hbm.at[pl.ds(wid * nch, nch)], idx_v)

        out_base = wid * b_per_w

        def start_gather(c):
            return pltpu.async_copy(
                table_hbm.at[idx_v.at[c]], bufs[c % _NBUF],
                gsems[c % _NBUF])

        gh = [None] * nch
        oh = [None] * nch
        gh[0] = start_gather(0)
        gh[1] = start_gather(1)
        for c in range(nch):
            if c + 2 < nch:
                if c - 1 >= 0:
                    oh[c - 1].wait()
                gh[c + 2] = start_gather(c + 2)
            gh[c].wait()
            oh[c] = pltpu.async_copy(
                bufs[c % _NBUF],
                out_hbm.at[pl.ds(out_base + c * _CHUNK, _CHUNK)],
                osems[c % _NBUF])
        for c in range(max(0, nch - _NBUF), nch):
            oh[c].wait()

    return gather_kernel


def _tc_onehot_body(pos_ref, tab_ref, out_ref, acc_ref):
    k = pl.program_id(1)

    @pl.when(k == 0)
    def _zero():
        acc_ref[...] = jnp.zeros_like(acc_ref)

    pos = pos_ref[0, 0, :]
    col = _K_BLK * k + lax.broadcasted_iota(
        jnp.int32, (_B_BLK, _K_BLK), 1)
    onehot = (pos[:, None] == col).astype(jnp.bfloat16)
    acc_ref[...] += jnp.dot(onehot, tab_ref[...],
                            preferred_element_type=jnp.float32)

    @pl.when(k == pl.num_programs(1) - 1)
    def _flush():
        out_ref[...] = acc_ref[...]


def _make_tc_gather(b_tc: int, n_rows: int, d_model: int):
    nb = b_tc // _B_BLK
    nk = n_rows // _K_BLK
    return pl.pallas_call(
        _tc_onehot_body,
        grid=(nb, nk),
        in_specs=[
            pl.BlockSpec((1, 1, _B_BLK), lambda b, k: (b, 0, 0)),
            pl.BlockSpec((_K_BLK, d_model), lambda b, k: (k, 0)),
        ],
        out_specs=pl.BlockSpec((_B_BLK, d_model), lambda b, k: (b, 0)),
        out_shape=jax.ShapeDtypeStruct((b_tc, d_model), jnp.float32),
        scratch_shapes=[pltpu.VMEM((_B_BLK, d_model), jnp.float32)],
    )


def kernel(positions, positional_embeddings):
    n_rows = positional_embeddings.shape[0]
    d_model = positional_embeddings.shape[-1]
    batch = positions.shape[0]
    b_tc = batch - _B_SC

    table = positional_embeddings.reshape(n_rows, d_model)
    table_bf16 = table.astype(jnp.bfloat16)

    idx_sc = positions[:_B_SC].reshape(_B_SC // _CHUNK, _CHUNK)
    pos_tc = positions[_B_SC:].reshape(b_tc // _B_BLK, 1, _B_BLK)

    sc_out = _make_sc_gather(batch, d_model)(idx_sc, table)
    return sc_out.reshape(batch, 1, d_model)


# speedup vs baseline: 1.9457x; 1.9457x over previous
import functools

import jax
import jax.numpy as jnp
from jax import lax
from jax.experimental import pallas as pl
from jax.experimental.pallas import tpu as pltpu
from jax.experimental.pallas import tpu_sc as plsc

_INFO = plsc.get_sparse_core_info()
_NC = _INFO.num_cores
_NS = _INFO.num_subcores
_NW = _NC * _NS

_CHUNK = 32
_NBUF = 3


def _make_sc_gather(batch: int, n_rows: int, d_model: int, chunk: int,
                    nbuf: int):
    b_per_w = batch // _NW
    nch = b_per_w // chunk

    mesh = plsc.VectorSubcoreMesh(core_axis_name="c", subcore_axis_name="s")

    @functools.partial(
        pl.kernel,
        mesh=mesh,
        out_type=jax.ShapeDtypeStruct((batch, 1, d_model), jnp.float32),
        scratch_types=[
            pltpu.VMEM((b_per_w,), jnp.int32),
            [pltpu.VMEM((chunk, 1, d_model), jnp.float32)] * nbuf,
            [pltpu.SemaphoreType.DMA] * nbuf,
            [pltpu.SemaphoreType.DMA] * nbuf,
        ],
    )
    def gather_kernel(pos_hbm, table_hbm, out_hbm, idx_v, bufs, gsems,
                      osems):
        wid = lax.axis_index("s") * _NC + lax.axis_index("c")
        out_base = wid * b_per_w
        pltpu.sync_copy(pos_hbm.at[pl.ds(out_base, b_per_w)], idx_v)

        def start_gather(c):
            return pltpu.async_copy(
                table_hbm.at[idx_v.at[pl.ds(c * chunk, chunk)]],
                bufs[c % nbuf], gsems[c % nbuf])

        gh = [None] * nch
        oh = [None] * nch
        for c in range(nbuf - 1):
            gh[c] = start_gather(c)
        for c in range(nch):
            if c + nbuf - 1 < nch:
                if c - 1 >= 0:
                    oh[c - 1].wait()
                gh[c + nbuf - 1] = start_gather(c + nbuf - 1)
            gh[c].wait()
            oh[c] = pltpu.async_copy(
                bufs[c % nbuf],
                out_hbm.at[pl.ds(out_base + c * chunk, chunk)],
                osems[c % nbuf])
        for c in range(max(0, nch - nbuf), nch):
            oh[c].wait()

    return gather_kernel


def kernel(positions, positional_embeddings):
    n_rows = positional_embeddings.shape[0]
    d_model = positional_embeddings.shape[-1]
    batch = positions.shape[0]
    return _make_sc_gather(batch, n_rows, d_model, _CHUNK, _NBUF)(
        positions, positional_embeddings)
